# Initial kernel scaffold; baseline (speedup 1.0000x reference)
import jax
import jax.numpy as jnp
from jax.experimental import pallas as pl


def kernel(x, edge_index, W1, a_src1, a_dst1, b1, W2, a_src2, a_dst2, b2):
    raise NotImplementedError("write your pallas kernel here")


# speedup vs baseline: 31.1200x; 31.1200x over previous
import functools

import jax
import jax.numpy as jnp
from jax import lax
from jax.experimental import pallas as pl
from jax.experimental.pallas import tpu as pltpu
from jax.experimental.pallas import tpu_sc as plsc

N = 10000
E = 320000
CH = 128
NHEAD1 = 8
ROWW = 144
DEN0 = 128

NTILES = 32
K = 128
ET = E + N
CHUNKS = -(-ET // (NTILES * K))
PER_TILE = CHUNKS * K
EPAD = PER_TILE * NTILES
NP_ = ((N + 1 + 127) // 128) * 128
RPT = NP_ // 16

_NBLK = 10
_BLK = N // _NBLK


def _pre_body(x_ref, w_ref, asg_ref, adg_ref, h_ref, es_ref, ed_ref):
    h = jnp.dot(x_ref[...], w_ref[...], preferred_element_type=jnp.float32,
                precision=lax.Precision.HIGHEST)
    h_ref[...] = h
    es_ref[...] = jnp.dot(h, asg_ref[...], preferred_element_type=jnp.float32,
                          precision=lax.Precision.HIGHEST)
    ed_ref[...] = jnp.dot(h, adg_ref[...], preferred_element_type=jnp.float32,
                          precision=lax.Precision.HIGHEST)


def _tc_pre(x, w, asg, adg):
    return pl.pallas_call(
        _pre_body,
        grid=(_NBLK,),
        in_specs=[
            pl.BlockSpec((_BLK, CH), lambda i: (i, 0)),
            pl.BlockSpec((CH, CH), lambda i: (0, 0)),
            pl.BlockSpec((CH, 16), lambda i: (0, 0)),
            pl.BlockSpec((CH, 16), lambda i: (0, 0)),
        ],
        out_specs=[
            pl.BlockSpec((_BLK, CH), lambda i: (i, 0)),
            pl.BlockSpec((_BLK, 16), lambda i: (i, 0)),
            pl.BlockSpec((_BLK, 16), lambda i: (i, 0)),
        ],
        out_shape=[
            jax.ShapeDtypeStruct((N, CH), jnp.float32),
            jax.ShapeDtypeStruct((N, 16), jnp.float32),
            jax.ShapeDtypeStruct((N, 16), jnp.float32),
        ],
    )(x, w, asg, adg)


def _mid_body(p_ref, q_ref, b_ref, w2_ref, asg_ref, adg_ref,
              h2_ref, es_ref, ed_ref):
    acc = p_ref[0] + p_ref[1]
    num = acc[:, :CH]
    den = jnp.dot(acc, q_ref[...], preferred_element_type=jnp.float32,
                  precision=lax.Precision.HIGHEST)
    out1 = num / (den + 1e-16) + b_ref[...]
    h1 = jnp.maximum(out1, 0.0)
    h2 = jnp.dot(h1, w2_ref[...], preferred_element_type=jnp.float32,
                 precision=lax.Precision.HIGHEST)
    h2_ref[...] = h2
    es_ref[...] = jnp.dot(h2, asg_ref[...], preferred_element_type=jnp.float32,
                          precision=lax.Precision.HIGHEST)
    ed_ref[...] = jnp.dot(h2, adg_ref[...], preferred_element_type=jnp.float32,
                          precision=lax.Precision.HIGHEST)


def _tc_mid(p, q, b, w2, asg, adg):
    return pl.pallas_call(
        _mid_body,
        grid=(_NBLK,),
        in_specs=[
            pl.BlockSpec((2, _BLK, ROWW), lambda i: (0, i, 0)),
            pl.BlockSpec((ROWW, CH), lambda i: (0, 0)),
            pl.BlockSpec((1, CH), lambda i: (0, 0)),
            pl.BlockSpec((CH, CH), lambda i: (0, 0)),
            pl.BlockSpec((CH, 16), lambda i: (0, 0)),
            pl.BlockSpec((CH, 16), lambda i: (0, 0)),
        ],
        out_specs=[
            pl.BlockSpec((_BLK, CH), lambda i: (i, 0)),
            pl.BlockSpec((_BLK, 16), lambda i: (i, 0)),
            pl.BlockSpec((_BLK, 16), lambda i: (i, 0)),
        ],
        out_shape=[
            jax.ShapeDtypeStruct((N, CH), jnp.float32),
            jax.ShapeDtypeStruct((N, 16), jnp.float32),
            jax.ShapeDtypeStruct((N, 16), jnp.float32),
        ],
    )(p, q, b, w2, asg, adg)


def _post_body(p_ref, q_ref, b_ref, o_ref):
    acc = p_ref[0] + p_ref[1]
    num = acc[:, :CH]
    den = jnp.dot(acc, q_ref[...], preferred_element_type=jnp.float32,
                  precision=lax.Precision.HIGHEST)
    o_ref[...] = num / (den + 1e-16) + b_ref[...]


def _tc_post(p, q, b):
    return pl.pallas_call(
        _post_body,
        grid=(_NBLK,),
        in_specs=[
            pl.BlockSpec((2, _BLK, ROWW), lambda i: (0, i, 0)),
            pl.BlockSpec((ROWW, CH), lambda i: (0, 0)),
            pl.BlockSpec((1, CH), lambda i: (0, 0)),
        ],
        out_specs=pl.BlockSpec((_BLK, CH), lambda i: (i, 0)),
        out_shape=jax.ShapeDtypeStruct((N, CH), jnp.float32),
    )(p, q, b)


def _make_edge_pass(heads):
    mesh = plsc.VectorSubcoreMesh(core_axis_name="c", subcore_axis_name="s")

    @functools.partial(
        pl.kernel,
        out_type=jax.ShapeDtypeStruct((2, NP_, ROWW), jnp.float32),
        mesh=mesh,
        compiler_params=pltpu.CompilerParams(
            use_tc_tiling_on_sc=False, needs_layout_passes=False),
        scratch_types=[
            pltpu.VMEM_SHARED((NP_, ROWW), jnp.float32),
            pltpu.VMEM((K,), jnp.int32),
            pltpu.VMEM((K,), jnp.int32),
            pltpu.VMEM((K, CH), jnp.float32),
            pltpu.VMEM((K, 16), jnp.float32),
            pltpu.VMEM((K, 16), jnp.float32),
            pltpu.VMEM((K, ROWW), jnp.float32),
            pltpu.SemaphoreType.DMA,
            pltpu.SemaphoreType.DMA,
            pltpu.SemaphoreType.DMA,
        ],
    )
    def edge_pass(h_hbm, es_hbm, ed_hbm, src_hbm, dst_hbm, z_hbm, out_hbm,
                  acc, src_v, dst_v, h_rows, es_rows, ed_rows, msg,
                  sem_h, sem_e, sem_d):
        cid = lax.axis_index("c")
        sid = lax.axis_index("s")
        wid = sid * 2 + cid

        pltpu.sync_copy(z_hbm, acc.at[pl.ds(sid * RPT, RPT)])
        plsc.subcore_barrier()

        e_base = wid * PER_TILE

        def chunk_body(i, carry):
            e0 = e_base + i * K
            pltpu.sync_copy(src_hbm.at[pl.ds(e0, K)], src_v)
            pltpu.sync_copy(dst_hbm.at[pl.ds(e0, K)], dst_v)
            cp_h = pltpu.async_copy(h_hbm.at[src_v], h_rows, sem_h)
            cp_e = pltpu.async_copy(es_hbm.at[src_v], es_rows, sem_e)
            cp_d = pltpu.async_copy(ed_hbm.at[dst_v], ed_rows, sem_d)
            cp_e.wait()
            cp_d.wait()

            iv = lax.iota(jnp.int32, 16)
            for g in range(K // 16):
                kidx = iv + g * 16
                for hd in range(heads):
                    col = jnp.full((16,), hd, jnp.int32)
                    s = (plsc.load_gather(es_rows, [kidx, col])
                         + plsc.load_gather(ed_rows, [kidx, col]))
                    s = jnp.where(s > 0, s, 0.2 * s)
                    w = jnp.exp(s)
                    plsc.store_scatter(
                        msg, [kidx, jnp.full((16,), DEN0 + hd, jnp.int32)], w)

            cp_h.wait()

            def edge_body(k, c2):
                wv = msg[k, pl.ds(DEN0, 16)]
                for hd in range(8):
                    ws = wv[hd % heads]
                    msg[k, pl.ds(hd * 16, 16)] = ws * h_rows[k, pl.ds(hd * 16, 16)]
                return c2

            lax.fori_loop(0, K, edge_body, 0)
            pltpu.sync_copy(msg, acc.at[dst_v], add=True)
            return carry

        lax.fori_loop(0, CHUNKS, chunk_body, 0)

        plsc.subcore_barrier()
        pltpu.sync_copy(acc.at[pl.ds(sid * RPT, RPT)],
                        out_hbm.at[cid, pl.ds(sid * RPT, RPT)])

    return edge_pass


_edge_pass8 = _make_edge_pass(NHEAD1)
_edge_pass1 = _make_edge_pass(1)


def kernel(x, edge_index, W1, a_src1, a_dst1, b1, W2, a_src2, a_dst2, b2):
    f32 = jnp.float32
    cols = jnp.arange(CH)
    grp = cols // 16

    asg1 = jnp.zeros((CH, 16), f32).at[cols, grp].set(a_src1.reshape(CH))
    adg1 = jnp.zeros((CH, 16), f32).at[cols, grp].set(a_dst1.reshape(CH))
    asg2 = jnp.zeros((CH, 16), f32).at[:, 0].set(a_src2[0])
    adg2 = jnp.zeros((CH, 16), f32).at[:, 0].set(a_dst2[0])

    q1 = jnp.zeros((ROWW, CH), f32).at[DEN0 + grp, cols].set(1.0)
    q2 = jnp.zeros((ROWW, CH), f32).at[DEN0, cols].set(1.0)

    loop = jnp.arange(N, dtype=jnp.int32)
    npad = EPAD - ET
    src = jnp.concatenate([edge_index[0], loop,
                           jnp.zeros((npad,), jnp.int32)])
    dst = jnp.concatenate([edge_index[1], loop,
                           jnp.full((npad,), N, jnp.int32)])
    zrows = jnp.zeros((RPT, ROWW), f32)
    padrows = jnp.zeros((NP_ - N, 16), f32)

    h1, es1, ed1 = _tc_pre(x, W1, asg1, adg1)
    ed1p = jnp.concatenate([ed1, padrows])
    p1 = _edge_pass8(h1, es1, ed1p, src, dst, zrows)

    h2, es2, ed2 = _tc_mid(p1, q1, b1.reshape(1, CH), W2, asg2, adg2)
    ed2p = jnp.concatenate([ed2, padrows])
    p2 = _edge_pass1(h2, es2, ed2p, src, dst, zrows)

    return _tc_post(p2, q2, b2.reshape(1, CH))


# speedup vs baseline: 39.8259x; 1.2797x over previous
import functools

import jax
import jax.numpy as jnp
from jax import lax
from jax.experimental import pallas as pl
from jax.experimental.pallas import tpu as pltpu
from jax.experimental.pallas import tpu_sc as plsc

N = 10000
E = 320000
CH = 128
NHEAD1 = 8
ROWW = 136
DEN0 = 128

NTILES = 32
K = 64
ET = E + N
NBUF = 2
CHUNKS = ((-(-ET // (NTILES * K)) + NBUF - 1) // NBUF) * NBUF
PER_TILE = CHUNKS * K
EPAD = PER_TILE * NTILES
NP_ = ((N + 1 + 127) // 128) * 128
RPT = NP_ // 16

_NBLK = 10
_BLK = N // _NBLK


def _pre_body(x_ref, w_ref, asg_ref, adg_ref, h_ref, es_ref, ed_ref):
    h = jnp.dot(x_ref[...], w_ref[...], preferred_element_type=jnp.float32,
                precision=lax.Precision.HIGHEST)
    h_ref[...] = h
    es_ref[...] = jnp.dot(h, asg_ref[...], preferred_element_type=jnp.float32,
                          precision=lax.Precision.HIGHEST)
    ed_ref[...] = jnp.dot(h, adg_ref[...], preferred_element_type=jnp.float32,
                          precision=lax.Precision.HIGHEST)


def _tc_pre(x, w, asg, adg):
    return pl.pallas_call(
        _pre_body,
        grid=(_NBLK,),
        in_specs=[
            pl.BlockSpec((_BLK, CH), lambda i: (i, 0)),
            pl.BlockSpec((CH, CH), lambda i: (0, 0)),
            pl.BlockSpec((CH, 16), lambda i: (0, 0)),
            pl.BlockSpec((CH, 16), lambda i: (0, 0)),
        ],
        out_specs=[
            pl.BlockSpec((_BLK, CH), lambda i: (i, 0)),
            pl.BlockSpec((_BLK, 16), lambda i: (i, 0)),
            pl.BlockSpec((_BLK, 16), lambda i: (i, 0)),
        ],
        out_shape=[
            jax.ShapeDtypeStruct((N, CH), jnp.float32),
            jax.ShapeDtypeStruct((N, 16), jnp.float32),
            jax.ShapeDtypeStruct((N, 16), jnp.float32),
        ],
    )(x, w, asg, adg)


def _mid_body(p_ref, q_ref, b_ref, w2_ref, asg_ref, adg_ref,
              h2_ref, es_ref, ed_ref):
    acc = p_ref[0] + p_ref[1]
    num = acc[:, :CH]
    den = jnp.dot(acc, q_ref[...], preferred_element_type=jnp.float32,
                  precision=lax.Precision.HIGHEST)
    out1 = num / (den + 1e-16) + b_ref[...]
    h1 = jnp.maximum(out1, 0.0)
    h2 = jnp.dot(h1, w2_ref[...], preferred_element_type=jnp.float32,
                 precision=lax.Precision.HIGHEST)
    h2_ref[...] = h2
    es_ref[...] = jnp.dot(h2, asg_ref[...], preferred_element_type=jnp.float32,
                          precision=lax.Precision.HIGHEST)
    ed_ref[...] = jnp.dot(h2, adg_ref[...], preferred_element_type=jnp.float32,
                          precision=lax.Precision.HIGHEST)


def _tc_mid(p, q, b, w2, asg, adg):
    return pl.pallas_call(
        _mid_body,
        grid=(_NBLK,),
        in_specs=[
            pl.BlockSpec((2, _BLK, ROWW), lambda i: (0, i, 0)),
            pl.BlockSpec((ROWW, CH), lambda i: (0, 0)),
            pl.BlockSpec((1, CH), lambda i: (0, 0)),
            pl.BlockSpec((CH, CH), lambda i: (0, 0)),
            pl.BlockSpec((CH, 16), lambda i: (0, 0)),
            pl.BlockSpec((CH, 16), lambda i: (0, 0)),
        ],
        out_specs=[
            pl.BlockSpec((_BLK, CH), lambda i: (i, 0)),
            pl.BlockSpec((_BLK, 16), lambda i: (i, 0)),
            pl.BlockSpec((_BLK, 16), lambda i: (i, 0)),
        ],
        out_shape=[
            jax.ShapeDtypeStruct((N, CH), jnp.float32),
            jax.ShapeDtypeStruct((N, 16), jnp.float32),
            jax.ShapeDtypeStruct((N, 16), jnp.float32),
        ],
    )(p, q, b, w2, asg, adg)


def _post_body(p_ref, q_ref, b_ref, o_ref):
    acc = p_ref[0] + p_ref[1]
    num = acc[:, :CH]
    den = jnp.dot(acc, q_ref[...], preferred_element_type=jnp.float32,
                  precision=lax.Precision.HIGHEST)
    o_ref[...] = num / (den + 1e-16) + b_ref[...]


def _tc_post(p, q, b):
    return pl.pallas_call(
        _post_body,
        grid=(_NBLK,),
        in_specs=[
            pl.BlockSpec((2, _BLK, ROWW), lambda i: (0, i, 0)),
            pl.BlockSpec((ROWW, CH), lambda i: (0, 0)),
            pl.BlockSpec((1, CH), lambda i: (0, 0)),
        ],
        out_specs=pl.BlockSpec((_BLK, CH), lambda i: (i, 0)),
        out_shape=jax.ShapeDtypeStruct((N, CH), jnp.float32),
    )(p, q, b)


def _make_edge_pass(heads):
    mesh = plsc.VectorSubcoreMesh(core_axis_name="c", subcore_axis_name="s")

    @functools.partial(
        pl.kernel,
        out_type=jax.ShapeDtypeStruct((2, NP_, ROWW), jnp.float32),
        mesh=mesh,
        compiler_params=pltpu.CompilerParams(
            use_tc_tiling_on_sc=False, needs_layout_passes=False),
        scratch_types=[
            pltpu.VMEM_SHARED((NP_, ROWW), jnp.float32),
            [pltpu.VMEM((K,), jnp.int32) for _ in range(NBUF)],
            [pltpu.VMEM((K,), jnp.int32) for _ in range(NBUF)],
            [pltpu.VMEM((K,), jnp.int32) for _ in range(NBUF)],
            [pltpu.VMEM((K, CH), jnp.float32) for _ in range(NBUF)],
            [pltpu.VMEM((K, 16), jnp.float32) for _ in range(NBUF)],
            [pltpu.VMEM((K, 16), jnp.float32) for _ in range(NBUF)],
            [pltpu.VMEM((K, ROWW), jnp.float32) for _ in range(NBUF)],
            [pltpu.SemaphoreType.DMA for _ in range(NBUF)],
            [pltpu.SemaphoreType.DMA for _ in range(NBUF)],
            [pltpu.SemaphoreType.DMA for _ in range(NBUF)],
            [pltpu.SemaphoreType.DMA for _ in range(NBUF)],
        ],
    )
    def edge_pass(h_hbm, es_hbm, ed_hbm, src_hbm, dst_hbm, z_hbm, out_hbm,
                  acc, src_v, dst_v, sdst, h_rows, es_rows, ed_rows, msg,
                  sem_i, sem_h, sem_e, sem_d):
        cid = lax.axis_index("c")
        sid = lax.axis_index("s")
        wid = sid * 2 + cid

        pltpu.sync_copy(z_hbm, acc.at[pl.ds(sid * RPT, RPT)])
        plsc.subcore_barrier()

        e_base = wid * PER_TILE

        def idx_descs(b, i):
            e0 = e_base + i * K
            return (pltpu.make_async_copy(src_hbm.at[pl.ds(e0, K)],
                                          src_v[b], sem_i[b]),
                    pltpu.make_async_copy(dst_hbm.at[pl.ds(e0, K)],
                                          dst_v[b], sem_i[b]))

        def issue_idx(b, i):
            for d in idx_descs(b, i):
                d.start()

        def issue_gathers(b, i):
            for d in idx_descs(b, i):
                d.wait()
            pltpu.async_copy(h_hbm.at[src_v[b]], h_rows[b], sem_h[b])
            pltpu.async_copy(es_hbm.at[src_v[b]], es_rows[b], sem_e[b])
            pltpu.async_copy(ed_hbm.at[dst_v[b]], ed_rows[b], sem_d[b])

        def consume_pre(b, i):
            pltpu.make_async_copy(
                es_hbm.at[src_v[b]], es_rows[b], sem_e[b]).wait()
            pltpu.make_async_copy(
                ed_hbm.at[dst_v[b]], ed_rows[b], sem_d[b]).wait()
            pltpu.make_async_copy(
                h_hbm.at[src_v[b]], h_rows[b], sem_h[b]).wait()

            for j in range(K // 16):
                sdst[b][pl.ds(j * 16, 16)] = dst_v[b][pl.ds(j * 16, 16)]

        def consume_main(b, i):
            mb = msg[b]
            eb = es_rows[b]
            db = ed_rows[b]
            hb = h_rows[b]
            iv = lax.iota(jnp.int32, 16)
            for g in range(K // 16):
                kidx = iv + g * 16
                for hd in range(heads):
                    col = jnp.full((16,), hd, jnp.int32)
                    s = (plsc.load_gather(eb, [kidx, col])
                         + plsc.load_gather(db, [kidx, col]))
                    s = jnp.where(s > 0, s, 0.2 * s)
                    w = jnp.exp(s)
                    plsc.store_scatter(
                        mb, [kidx, jnp.full((16,), DEN0 + hd, jnp.int32)], w)

            def edge_body(k, c2):
                wv = mb[k, pl.ds(DEN0 - 8, 16)]
                for hd in range(8):
                    ws = wv[8 + (hd % heads)]
                    mb[k, pl.ds(hd * 16, 16)] = ws * hb[k, pl.ds(hd * 16, 16)]
                return c2

            lax.fori_loop(0, K, edge_body, 0)
            pltpu.sync_copy(mb, acc.at[sdst[b]], add=True)

        issue_idx(0, 0)
        issue_gathers(0, 0)
        issue_idx(1, 1)

        def chunk_body(i2, carry):
            for b in range(NBUF):
                i = i2 * NBUF + b
                nb = (b + 1) % NBUF

                consume_pre(b, i)

                @pl.when(i + 1 < CHUNKS)
                def _():
                    issue_gathers(nb, i + 1)

                @pl.when(i + 2 < CHUNKS)
                def _():
                    issue_idx(b, i + 2)

                consume_main(b, i)
            return carry

        lax.fori_loop(0, CHUNKS // NBUF, chunk_body, 0)

        plsc.subcore_barrier()
        pltpu.sync_copy(acc.at[pl.ds(sid * RPT, RPT)],
                        out_hbm.at[cid, pl.ds(sid * RPT, RPT)])

    return edge_pass


_edge_pass8 = _make_edge_pass(NHEAD1)
_edge_pass1 = _make_edge_pass(1)


def kernel(x, edge_index, W1, a_src1, a_dst1, b1, W2, a_src2, a_dst2, b2):
    f32 = jnp.float32
    cols = jnp.arange(CH)
    grp = cols // 16

    asg1 = jnp.zeros((CH, 16), f32).at[cols, grp].set(a_src1.reshape(CH))
    adg1 = jnp.zeros((CH, 16), f32).at[cols, grp].set(a_dst1.reshape(CH))
    asg2 = jnp.zeros((CH, 16), f32).at[:, 0].set(a_src2[0])
    adg2 = jnp.zeros((CH, 16), f32).at[:, 0].set(a_dst2[0])

    q1 = jnp.zeros((ROWW, CH), f32).at[DEN0 + grp, cols].set(1.0)
    q2 = jnp.zeros((ROWW, CH), f32).at[DEN0, cols].set(1.0)

    loop = jnp.arange(N, dtype=jnp.int32)
    npad = EPAD - ET
    src = jnp.concatenate([edge_index[0], loop,
                           jnp.zeros((npad,), jnp.int32)])
    dst = jnp.concatenate([edge_index[1], loop,
                           jnp.full((npad,), N, jnp.int32)])
    zrows = jnp.zeros((RPT, ROWW), f32)
    padrows = jnp.zeros((NP_ - N, 16), f32)

    h1, es1, ed1 = _tc_pre(x, W1, asg1, adg1)
    ed1p = jnp.concatenate([ed1, padrows])
    p1 = _edge_pass8(h1, es1, ed1p, src, dst, zrows)

    h2, es2, ed2 = _tc_mid(p1, q1, b1.reshape(1, CH), W2, asg2, adg2)
    ed2p = jnp.concatenate([ed2, padrows])
    p2 = _edge_pass1(h2, es2, ed2p, src, dst, zrows)

    return _tc_post(p2, q2, b2.reshape(1, CH))


# speedup vs baseline: 64.5324x; 1.6204x over previous
import functools

import jax
import jax.numpy as jnp
from jax import lax
from jax.experimental import pallas as pl
from jax.experimental.pallas import tpu as pltpu
from jax.experimental.pallas import tpu_sc as plsc

N = 10000
E = 320000
CH = 128
NHEAD1 = 8
ROWW = 136
DEN0 = 128

NTILES = 32
K = 64
ET = E + N
NBUF = 2
CHUNKS = ((-(-ET // (NTILES * K)) + NBUF - 1) // NBUF) * NBUF
PER_TILE = CHUNKS * K
EPAD = PER_TILE * NTILES
NP_ = ((N + 1 + 127) // 128) * 128
RPT = NP_ // 16

_NBLK = 10
_BLK = N // _NBLK


def _pre_body(x_ref, w_ref, asg_ref, adg_ref, h_ref, es_ref, ed_ref):
    h = jnp.dot(x_ref[...], w_ref[...], preferred_element_type=jnp.float32,
                precision=lax.Precision.HIGHEST)
    h_ref[...] = h
    es_ref[...] = jnp.dot(h, asg_ref[...], preferred_element_type=jnp.float32,
                          precision=lax.Precision.HIGHEST)
    ed_ref[...] = jnp.dot(h, adg_ref[...], preferred_element_type=jnp.float32,
                          precision=lax.Precision.HIGHEST)


def _tc_pre(x, w, asg, adg):
    return pl.pallas_call(
        _pre_body,
        grid=(_NBLK,),
        in_specs=[
            pl.BlockSpec((_BLK, CH), lambda i: (i, 0)),
            pl.BlockSpec((CH, CH), lambda i: (0, 0)),
            pl.BlockSpec((CH, 16), lambda i: (0, 0)),
            pl.BlockSpec((CH, 16), lambda i: (0, 0)),
        ],
        out_specs=[
            pl.BlockSpec((_BLK, CH), lambda i: (i, 0)),
            pl.BlockSpec((_BLK, 16), lambda i: (i, 0)),
            pl.BlockSpec((_BLK, 16), lambda i: (i, 0)),
        ],
        out_shape=[
            jax.ShapeDtypeStruct((N, CH), jnp.float32),
            jax.ShapeDtypeStruct((N, 16), jnp.float32),
            jax.ShapeDtypeStruct((N, 16), jnp.float32),
        ],
    )(x, w, asg, adg)


def _mid_body(p_ref, q_ref, b_ref, w2_ref, asg_ref, adg_ref,
              h2_ref, es_ref, ed_ref):
    acc = p_ref[0] + p_ref[1]
    num = acc[:, :CH]
    den = jnp.dot(acc, q_ref[...], preferred_element_type=jnp.float32,
                  precision=lax.Precision.HIGHEST)
    out1 = num / (den + 1e-16) + b_ref[...]
    h1 = jnp.maximum(out1, 0.0)
    h2 = jnp.dot(h1, w2_ref[...], preferred_element_type=jnp.float32,
                 precision=lax.Precision.HIGHEST)
    h2_ref[...] = h2
    es_ref[...] = jnp.dot(h2, asg_ref[...], preferred_element_type=jnp.float32,
                          precision=lax.Precision.HIGHEST)
    ed_ref[...] = jnp.dot(h2, adg_ref[...], preferred_element_type=jnp.float32,
                          precision=lax.Precision.HIGHEST)


def _tc_mid(p, q, b, w2, asg, adg):
    return pl.pallas_call(
        _mid_body,
        grid=(_NBLK,),
        in_specs=[
            pl.BlockSpec((2, _BLK, ROWW), lambda i: (0, i, 0)),
            pl.BlockSpec((ROWW, CH), lambda i: (0, 0)),
            pl.BlockSpec((1, CH), lambda i: (0, 0)),
            pl.BlockSpec((CH, CH), lambda i: (0, 0)),
            pl.BlockSpec((CH, 16), lambda i: (0, 0)),
            pl.BlockSpec((CH, 16), lambda i: (0, 0)),
        ],
        out_specs=[
            pl.BlockSpec((_BLK, CH), lambda i: (i, 0)),
            pl.BlockSpec((_BLK, 16), lambda i: (i, 0)),
            pl.BlockSpec((_BLK, 16), lambda i: (i, 0)),
        ],
        out_shape=[
            jax.ShapeDtypeStruct((N, CH), jnp.float32),
            jax.ShapeDtypeStruct((N, 16), jnp.float32),
            jax.ShapeDtypeStruct((N, 16), jnp.float32),
        ],
    )(p, q, b, w2, asg, adg)


def _post_body(p_ref, q_ref, b_ref, o_ref):
    acc = p_ref[0] + p_ref[1]
    num = acc[:, :CH]
    den = jnp.dot(acc, q_ref[...], preferred_element_type=jnp.float32,
                  precision=lax.Precision.HIGHEST)
    o_ref[...] = num / (den + 1e-16) + b_ref[...]


def _tc_post(p, q, b):
    return pl.pallas_call(
        _post_body,
        grid=(_NBLK,),
        in_specs=[
            pl.BlockSpec((2, _BLK, ROWW), lambda i: (0, i, 0)),
            pl.BlockSpec((ROWW, CH), lambda i: (0, 0)),
            pl.BlockSpec((1, CH), lambda i: (0, 0)),
        ],
        out_specs=pl.BlockSpec((_BLK, CH), lambda i: (i, 0)),
        out_shape=jax.ShapeDtypeStruct((N, CH), jnp.float32),
    )(p, q, b)


def _make_edge_pass(heads):
    mesh = plsc.VectorSubcoreMesh(core_axis_name="c", subcore_axis_name="s")

    @functools.partial(
        pl.kernel,
        out_type=jax.ShapeDtypeStruct((2, NP_, ROWW), jnp.float32),
        mesh=mesh,
        compiler_params=pltpu.CompilerParams(
            use_tc_tiling_on_sc=False, needs_layout_passes=False),
        scratch_types=[
            pltpu.VMEM_SHARED((NP_, ROWW), jnp.float32),
            [pltpu.VMEM((K,), jnp.int32) for _ in range(NBUF)],
            [pltpu.VMEM((K,), jnp.int32) for _ in range(NBUF)],
            [pltpu.VMEM((K,), jnp.int32) for _ in range(NBUF)],
            [pltpu.VMEM((K, CH), jnp.float32) for _ in range(NBUF)],
            [pltpu.VMEM((K, 16), jnp.float32) for _ in range(NBUF)],
            [pltpu.VMEM((K, 16), jnp.float32) for _ in range(NBUF)],
            [pltpu.VMEM((K, ROWW), jnp.float32) for _ in range(NBUF)],
            [pltpu.SemaphoreType.DMA for _ in range(NBUF)],
            [pltpu.SemaphoreType.DMA for _ in range(NBUF)],
            [pltpu.SemaphoreType.DMA for _ in range(NBUF)],
            [pltpu.SemaphoreType.DMA for _ in range(NBUF)],
        ],
    )
    def edge_pass(h_hbm, es_hbm, ed_hbm, src_hbm, dst_hbm, z_hbm, out_hbm,
                  acc, src_v, dst_v, sdst, h_rows, es_rows, ed_rows, msg,
                  sem_i, sem_h, sem_e, sem_d):
        cid = lax.axis_index("c")
        sid = lax.axis_index("s")
        wid = sid * 2 + cid

        pltpu.sync_copy(z_hbm, acc.at[pl.ds(sid * RPT, RPT)])
        plsc.subcore_barrier()

        e_base = wid * PER_TILE

        def idx_descs(b, i):
            e0 = e_base + i * K
            return (pltpu.make_async_copy(src_hbm.at[pl.ds(e0, K)],
                                          src_v[b], sem_i[b]),
                    pltpu.make_async_copy(dst_hbm.at[pl.ds(e0, K)],
                                          dst_v[b], sem_i[b]))

        def issue_idx(b, i):
            for d in idx_descs(b, i):
                d.start()

        def issue_gathers(b, i):
            for d in idx_descs(b, i):
                d.wait()
            pltpu.async_copy(h_hbm.at[src_v[b]], h_rows[b], sem_h[b])
            pltpu.async_copy(es_hbm.at[src_v[b]], es_rows[b], sem_e[b])
            pltpu.async_copy(ed_hbm.at[dst_v[b]], ed_rows[b], sem_d[b])

        def consume_pre(b, i):
            pltpu.make_async_copy(
                es_hbm.at[src_v[b]], es_rows[b], sem_e[b]).wait()
            pltpu.make_async_copy(
                ed_hbm.at[dst_v[b]], ed_rows[b], sem_d[b]).wait()
            pltpu.make_async_copy(
                h_hbm.at[src_v[b]], h_rows[b], sem_h[b]).wait()

            for j in range(K // 16):
                sdst[b][pl.ds(j * 16, 16)] = dst_v[b][pl.ds(j * 16, 16)]

        def consume_main(b, i):
            mb = msg[b]
            eb = es_rows[b]
            db = ed_rows[b]
            hb = h_rows[b]
            iv = lax.iota(jnp.int32, 16)
            for g in range(K // 16):
                kidx = iv + g * 16
                for hd in range(heads):
                    col = jnp.full((16,), hd, jnp.int32)
                    s = (plsc.load_gather(eb, [kidx, col])
                         + plsc.load_gather(db, [kidx, col]))
                    s = jnp.where(s > 0, s, 0.2 * s)
                    w = jnp.exp(s)
                    plsc.store_scatter(
                        mb, [kidx, jnp.full((16,), DEN0 + hd, jnp.int32)], w)

            @plsc.parallel_loop(0, K, step=1, unroll=4)
            def edge_body(k):
                wv = mb[k, pl.ds(DEN0 - 8, 16)]
                for hd in range(8):
                    ws = wv[8 + (hd % heads)]
                    mb[k, pl.ds(hd * 16, 16)] = ws * hb[k, pl.ds(hd * 16, 16)]

            pltpu.sync_copy(mb, acc.at[sdst[b]], add=True)

        issue_idx(0, 0)
        issue_gathers(0, 0)
        issue_idx(1, 1)

        def chunk_body(i2, carry):
            for b in range(NBUF):
                i = i2 * NBUF + b
                nb = (b + 1) % NBUF

                consume_pre(b, i)

                @pl.when(i + 1 < CHUNKS)
                def _():
                    issue_gathers(nb, i + 1)

                @pl.when(i + 2 < CHUNKS)
                def _():
                    issue_idx(b, i + 2)

                consume_main(b, i)
            return carry

        lax.fori_loop(0, CHUNKS // NBUF, chunk_body, 0)

        plsc.subcore_barrier()
        pltpu.sync_copy(acc.at[pl.ds(sid * RPT, RPT)],
                        out_hbm.at[cid, pl.ds(sid * RPT, RPT)])

    return edge_pass


_edge_pass8 = _make_edge_pass(NHEAD1)
_edge_pass1 = _make_edge_pass(1)


def kernel(x, edge_index, W1, a_src1, a_dst1, b1, W2, a_src2, a_dst2, b2):
    f32 = jnp.float32
    cols = jnp.arange(CH)
    grp = cols // 16

    asg1 = jnp.zeros((CH, 16), f32).at[cols, grp].set(a_src1.reshape(CH))
    adg1 = jnp.zeros((CH, 16), f32).at[cols, grp].set(a_dst1.reshape(CH))
    asg2 = jnp.zeros((CH, 16), f32).at[:, 0].set(a_src2[0])
    adg2 = jnp.zeros((CH, 16), f32).at[:, 0].set(a_dst2[0])

    q1 = jnp.zeros((ROWW, CH), f32).at[DEN0 + grp, cols].set(1.0)
    q2 = jnp.zeros((ROWW, CH), f32).at[DEN0, cols].set(1.0)

    loop = jnp.arange(N, dtype=jnp.int32)
    npad = EPAD - ET
    src = jnp.concatenate([edge_index[0], loop,
                           jnp.zeros((npad,), jnp.int32)])
    dst = jnp.concatenate([edge_index[1], loop,
                           jnp.full((npad,), N, jnp.int32)])
    zrows = jnp.zeros((RPT, ROWW), f32)
    padrows = jnp.zeros((NP_ - N, 16), f32)

    h1, es1, ed1 = _tc_pre(x, W1, asg1, adg1)
    ed1p = jnp.concatenate([ed1, padrows])
    p1 = _edge_pass8(h1, es1, ed1p, src, dst, zrows)

    h2, es2, ed2 = _tc_mid(p1, q1, b1.reshape(1, CH), W2, asg2, adg2)
    ed2p = jnp.concatenate([ed2, padrows])
    p2 = _edge_pass1(h2, es2, ed2p, src, dst, zrows)

    return _tc_post(p2, q2, b2.reshape(1, CH))


# speedup vs baseline: 64.9088x; 1.0058x over previous
import functools

import jax
import jax.numpy as jnp
from jax import lax
from jax.experimental import pallas as pl
from jax.experimental.pallas import tpu as pltpu
from jax.experimental.pallas import tpu_sc as plsc

N = 10000
E = 320000
CH = 128
NHEAD1 = 8
ROWW = 136
DEN0 = 128

NTILES = 32
K = 64
ET = E + N
NBUF = 2
CHUNKS = ((-(-ET // (NTILES * K)) + NBUF - 1) // NBUF) * NBUF
PER_TILE = CHUNKS * K
EPAD = PER_TILE * NTILES
NP_ = ((N + 1 + 127) // 128) * 128
RPT = NP_ // 16

_NBLK = 10
_BLK = N // _NBLK


def _pre_body(x_ref, w_ref, asg_ref, adg_ref, h_ref, es_ref, ed_ref):
    h = jnp.dot(x_ref[...], w_ref[...], preferred_element_type=jnp.float32,
                precision=lax.Precision.HIGHEST)
    h_ref[...] = h
    es_ref[...] = jnp.dot(h, asg_ref[...], preferred_element_type=jnp.float32,
                          precision=lax.Precision.HIGHEST)
    ed_ref[...] = jnp.dot(h, adg_ref[...], preferred_element_type=jnp.float32,
                          precision=lax.Precision.HIGHEST)


def _tc_pre(x, w, asg, adg):
    return pl.pallas_call(
        _pre_body,
        grid=(_NBLK,),
        in_specs=[
            pl.BlockSpec((_BLK, CH), lambda i: (i, 0)),
            pl.BlockSpec((CH, CH), lambda i: (0, 0)),
            pl.BlockSpec((CH, 16), lambda i: (0, 0)),
            pl.BlockSpec((CH, 16), lambda i: (0, 0)),
        ],
        out_specs=[
            pl.BlockSpec((_BLK, CH), lambda i: (i, 0)),
            pl.BlockSpec((_BLK, 16), lambda i: (i, 0)),
            pl.BlockSpec((_BLK, 16), lambda i: (i, 0)),
        ],
        out_shape=[
            jax.ShapeDtypeStruct((N, CH), jnp.float32),
            jax.ShapeDtypeStruct((N, 16), jnp.float32),
            jax.ShapeDtypeStruct((N, 16), jnp.float32),
        ],
    )(x, w, asg, adg)


def _mid_body(p_ref, q_ref, b_ref, w2_ref, asg_ref, adg_ref,
              h2_ref, es_ref, ed_ref):
    acc = p_ref[0] + p_ref[1]
    num = acc[:, :CH]
    den = jnp.dot(acc, q_ref[...], preferred_element_type=jnp.float32,
                  precision=lax.Precision.HIGHEST)
    out1 = num / (den + 1e-16) + b_ref[...]
    h1 = jnp.maximum(out1, 0.0)
    h2 = jnp.dot(h1, w2_ref[...], preferred_element_type=jnp.float32,
                 precision=lax.Precision.HIGHEST)
    h2_ref[...] = h2
    es_ref[...] = jnp.dot(h2, asg_ref[...], preferred_element_type=jnp.float32,
                          precision=lax.Precision.HIGHEST)
    ed_ref[...] = jnp.dot(h2, adg_ref[...], preferred_element_type=jnp.float32,
                          precision=lax.Precision.HIGHEST)


def _tc_mid(p, q, b, w2, asg, adg):
    return pl.pallas_call(
        _mid_body,
        grid=(_NBLK,),
        in_specs=[
            pl.BlockSpec((2, _BLK, ROWW), lambda i: (0, i, 0)),
            pl.BlockSpec((ROWW, CH), lambda i: (0, 0)),
            pl.BlockSpec((1, CH), lambda i: (0, 0)),
            pl.BlockSpec((CH, CH), lambda i: (0, 0)),
            pl.BlockSpec((CH, 16), lambda i: (0, 0)),
            pl.BlockSpec((CH, 16), lambda i: (0, 0)),
        ],
        out_specs=[
            pl.BlockSpec((_BLK, CH), lambda i: (i, 0)),
            pl.BlockSpec((_BLK, 16), lambda i: (i, 0)),
            pl.BlockSpec((_BLK, 16), lambda i: (i, 0)),
        ],
        out_shape=[
            jax.ShapeDtypeStruct((N, CH), jnp.float32),
            jax.ShapeDtypeStruct((N, 16), jnp.float32),
            jax.ShapeDtypeStruct((N, 16), jnp.float32),
        ],
    )(p, q, b, w2, asg, adg)


def _post_body(p_ref, q_ref, b_ref, o_ref):
    acc = p_ref[0] + p_ref[1]
    num = acc[:, :CH]
    den = jnp.dot(acc, q_ref[...], preferred_element_type=jnp.float32,
                  precision=lax.Precision.HIGHEST)
    o_ref[...] = num / (den + 1e-16) + b_ref[...]


def _tc_post(p, q, b):
    return pl.pallas_call(
        _post_body,
        grid=(_NBLK,),
        in_specs=[
            pl.BlockSpec((2, _BLK, ROWW), lambda i: (0, i, 0)),
            pl.BlockSpec((ROWW, CH), lambda i: (0, 0)),
            pl.BlockSpec((1, CH), lambda i: (0, 0)),
        ],
        out_specs=pl.BlockSpec((_BLK, CH), lambda i: (i, 0)),
        out_shape=jax.ShapeDtypeStruct((N, CH), jnp.float32),
    )(p, q, b)


def _make_edge_pass(heads):
    mesh = plsc.VectorSubcoreMesh(core_axis_name="c", subcore_axis_name="s")

    @functools.partial(
        pl.kernel,
        out_type=jax.ShapeDtypeStruct((2, NP_, ROWW), jnp.float32),
        mesh=mesh,
        compiler_params=pltpu.CompilerParams(
            use_tc_tiling_on_sc=False, needs_layout_passes=False),
        scratch_types=[
            pltpu.VMEM_SHARED((NP_, ROWW), jnp.float32),
            [pltpu.VMEM((K,), jnp.int32) for _ in range(NBUF)],
            [pltpu.VMEM((K,), jnp.int32) for _ in range(NBUF)],
            [pltpu.VMEM((K,), jnp.int32) for _ in range(NBUF)],
            [pltpu.VMEM((K, CH), jnp.float32) for _ in range(NBUF)],
            [pltpu.VMEM((K, 16), jnp.float32) for _ in range(NBUF)],
            [pltpu.VMEM((K, 16), jnp.float32) for _ in range(NBUF)],
            [pltpu.VMEM((K, ROWW), jnp.float32) for _ in range(NBUF)],
            [pltpu.SemaphoreType.DMA for _ in range(NBUF)],
            [pltpu.SemaphoreType.DMA for _ in range(NBUF)],
            [pltpu.SemaphoreType.DMA for _ in range(NBUF)],
            [pltpu.SemaphoreType.DMA for _ in range(NBUF)],
        ],
    )
    def edge_pass(h_hbm, es_hbm, ed_hbm, src_hbm, dst_hbm, z_hbm, out_hbm,
                  acc, src_v, dst_v, sdst, h_rows, es_rows, ed_rows, msg,
                  sem_i, sem_h, sem_e, sem_d):
        cid = lax.axis_index("c")
        sid = lax.axis_index("s")
        wid = sid * 2 + cid

        pltpu.sync_copy(z_hbm, acc.at[pl.ds(sid * RPT, RPT)])
        plsc.subcore_barrier()

        e_base = wid * PER_TILE

        def idx_descs(b, i):
            e0 = e_base + i * K
            return (pltpu.make_async_copy(src_hbm.at[pl.ds(e0, K)],
                                          src_v[b], sem_i[b]),
                    pltpu.make_async_copy(dst_hbm.at[pl.ds(e0, K)],
                                          dst_v[b], sem_i[b]))

        def issue_idx(b, i):
            for d in idx_descs(b, i):
                d.start()

        def issue_gathers(b, i):
            for d in idx_descs(b, i):
                d.wait()
            pltpu.async_copy(h_hbm.at[src_v[b]], h_rows[b], sem_h[b])
            pltpu.async_copy(es_hbm.at[src_v[b]], es_rows[b], sem_e[b])
            pltpu.async_copy(ed_hbm.at[dst_v[b]], ed_rows[b], sem_d[b])

        def consume_pre(b, i):
            pltpu.make_async_copy(
                es_hbm.at[src_v[b]], es_rows[b], sem_e[b]).wait()
            pltpu.make_async_copy(
                ed_hbm.at[dst_v[b]], ed_rows[b], sem_d[b]).wait()
            pltpu.make_async_copy(
                h_hbm.at[src_v[b]], h_rows[b], sem_h[b]).wait()

            for j in range(K // 16):
                sdst[b][pl.ds(j * 16, 16)] = dst_v[b][pl.ds(j * 16, 16)]

        def consume_main(b, i):
            mb = msg[b]
            eb = es_rows[b]
            db = ed_rows[b]
            hb = h_rows[b]
            iv = lax.iota(jnp.int32, 16)
            for g in range(K // 16):
                kidx = iv + g * 16
                for hd in range(heads):
                    col = jnp.full((16,), hd, jnp.int32)
                    s = (plsc.load_gather(eb, [kidx, col])
                         + plsc.load_gather(db, [kidx, col]))
                    s = jnp.where(s > 0, s, 0.2 * s)
                    w = jnp.exp(s)
                    plsc.store_scatter(
                        mb, [kidx, jnp.full((16,), DEN0 + hd, jnp.int32)], w)

            @plsc.parallel_loop(0, K, step=1, unroll=8)
            def edge_body(k):
                wv = mb[k, pl.ds(DEN0 - 8, 16)]
                for hd in range(8):
                    ws = wv[8 + (hd % heads)]
                    mb[k, pl.ds(hd * 16, 16)] = ws * hb[k, pl.ds(hd * 16, 16)]

            pltpu.sync_copy(mb, acc.at[sdst[b]], add=True)

        issue_idx(0, 0)
        issue_gathers(0, 0)
        issue_idx(1, 1)

        def chunk_body(i2, carry):
            for b in range(NBUF):
                i = i2 * NBUF + b
                nb = (b + 1) % NBUF

                consume_pre(b, i)

                @pl.when(i + 1 < CHUNKS)
                def _():
                    issue_gathers(nb, i + 1)

                @pl.when(i + 2 < CHUNKS)
                def _():
                    issue_idx(b, i + 2)

                consume_main(b, i)
            return carry

        lax.fori_loop(0, CHUNKS // NBUF, chunk_body, 0)

        plsc.subcore_barrier()
        pltpu.sync_copy(acc.at[pl.ds(sid * RPT, RPT)],
                        out_hbm.at[cid, pl.ds(sid * RPT, RPT)])

    return edge_pass


_edge_pass8 = _make_edge_pass(NHEAD1)
_edge_pass1 = _make_edge_pass(1)


def kernel(x, edge_index, W1, a_src1, a_dst1, b1, W2, a_src2, a_dst2, b2):
    f32 = jnp.float32
    cols = jnp.arange(CH)
    grp = cols // 16

    asg1 = jnp.zeros((CH, 16), f32).at[cols, grp].set(a_src1.reshape(CH))
    adg1 = jnp.zeros((CH, 16), f32).at[cols, grp].set(a_dst1.reshape(CH))
    asg2 = jnp.zeros((CH, 16), f32).at[:, 0].set(a_src2[0])
    adg2 = jnp.zeros((CH, 16), f32).at[:, 0].set(a_dst2[0])

    q1 = jnp.zeros((ROWW, CH), f32).at[DEN0 + grp, cols].set(1.0)
    q2 = jnp.zeros((ROWW, CH), f32).at[DEN0, cols].set(1.0)

    loop = jnp.arange(N, dtype=jnp.int32)
    npad = EPAD - ET
    src = jnp.concatenate([edge_index[0], loop,
                           jnp.zeros((npad,), jnp.int32)])
    dst = jnp.concatenate([edge_index[1], loop,
                           jnp.full((npad,), N, jnp.int32)])
    zrows = jnp.zeros((RPT, ROWW), f32)
    padrows = jnp.zeros((NP_ - N, 16), f32)

    h1, es1, ed1 = _tc_pre(x, W1, asg1, adg1)
    ed1p = jnp.concatenate([ed1, padrows])
    p1 = _edge_pass8(h1, es1, ed1p, src, dst, zrows)

    h2, es2, ed2 = _tc_mid(p1, q1, b1.reshape(1, CH), W2, asg2, adg2)
    ed2p = jnp.concatenate([ed2, padrows])
    p2 = _edge_pass1(h2, es2, ed2p, src, dst, zrows)

    return _tc_post(p2, q2, b2.reshape(1, CH))


# speedup vs baseline: 68.8392x; 1.0606x over previous
import functools

import jax
import jax.numpy as jnp
from jax import lax
from jax.experimental import pallas as pl
from jax.experimental.pallas import tpu as pltpu
from jax.experimental.pallas import tpu_sc as plsc

N = 10000
E = 320000
CH = 128
NHEAD1 = 8
ROWW = 136
DEN0 = 128

NTILES = 32
K = 64
ET = E + N
NBUF = 2
CHUNKS = ((-(-ET // (NTILES * K)) + NBUF - 1) // NBUF) * NBUF
PER_TILE = CHUNKS * K
EPAD = PER_TILE * NTILES
NP_ = ((N + 1 + 127) // 128) * 128
RPT = NP_ // 16

_NBLK = 10
_BLK = N // _NBLK


def _pre_body(x_ref, w_ref, asg_ref, adg_ref, h_ref, es_ref, ed_ref):
    h = jnp.dot(x_ref[...], w_ref[...], preferred_element_type=jnp.float32,
                precision=lax.Precision.HIGHEST)
    h_ref[...] = h
    es_ref[...] = jnp.dot(h, asg_ref[...], preferred_element_type=jnp.float32,
                          precision=lax.Precision.HIGHEST)
    ed_ref[...] = jnp.dot(h, adg_ref[...], preferred_element_type=jnp.float32,
                          precision=lax.Precision.HIGHEST)


def _tc_pre(x, w, asg, adg):
    return pl.pallas_call(
        _pre_body,
        grid=(_NBLK,),
        in_specs=[
            pl.BlockSpec((_BLK, CH), lambda i: (i, 0)),
            pl.BlockSpec((CH, CH), lambda i: (0, 0)),
            pl.BlockSpec((CH, 16), lambda i: (0, 0)),
            pl.BlockSpec((CH, 16), lambda i: (0, 0)),
        ],
        out_specs=[
            pl.BlockSpec((_BLK, CH), lambda i: (i, 0)),
            pl.BlockSpec((_BLK, 16), lambda i: (i, 0)),
            pl.BlockSpec((_BLK, 16), lambda i: (i, 0)),
        ],
        out_shape=[
            jax.ShapeDtypeStruct((N, CH), jnp.float32),
            jax.ShapeDtypeStruct((N, 16), jnp.float32),
            jax.ShapeDtypeStruct((N, 16), jnp.float32),
        ],
    )(x, w, asg, adg)


def _mid_body(p_ref, q_ref, b_ref, w2_ref, asg_ref, adg_ref,
              h2_ref, es_ref, ed_ref):
    acc = p_ref[0] + p_ref[1]
    num = acc[:, :CH]
    den = jnp.dot(acc, q_ref[...], preferred_element_type=jnp.float32,
                  precision=lax.Precision.HIGHEST)
    out1 = num / (den + 1e-16) + b_ref[...]
    h1 = jnp.maximum(out1, 0.0)
    h2 = jnp.dot(h1, w2_ref[...], preferred_element_type=jnp.float32,
                 precision=lax.Precision.HIGHEST)
    h2_ref[...] = h2
    es_ref[...] = jnp.dot(h2, asg_ref[...], preferred_element_type=jnp.float32,
                          precision=lax.Precision.HIGHEST)
    ed_ref[...] = jnp.dot(h2, adg_ref[...], preferred_element_type=jnp.float32,
                          precision=lax.Precision.HIGHEST)


def _tc_mid(p, q, b, w2, asg, adg):
    return pl.pallas_call(
        _mid_body,
        grid=(_NBLK,),
        in_specs=[
            pl.BlockSpec((2, _BLK, ROWW), lambda i: (0, i, 0)),
            pl.BlockSpec((ROWW, CH), lambda i: (0, 0)),
            pl.BlockSpec((1, CH), lambda i: (0, 0)),
            pl.BlockSpec((CH, CH), lambda i: (0, 0)),
            pl.BlockSpec((CH, 16), lambda i: (0, 0)),
            pl.BlockSpec((CH, 16), lambda i: (0, 0)),
        ],
        out_specs=[
            pl.BlockSpec((_BLK, CH), lambda i: (i, 0)),
            pl.BlockSpec((_BLK, 16), lambda i: (i, 0)),
            pl.BlockSpec((_BLK, 16), lambda i: (i, 0)),
        ],
        out_shape=[
            jax.ShapeDtypeStruct((N, CH), jnp.float32),
            jax.ShapeDtypeStruct((N, 16), jnp.float32),
            jax.ShapeDtypeStruct((N, 16), jnp.float32),
        ],
    )(p, q, b, w2, asg, adg)


def _post_body(p_ref, q_ref, b_ref, o_ref):
    acc = p_ref[0] + p_ref[1]
    num = acc[:, :CH]
    den = jnp.dot(acc, q_ref[...], preferred_element_type=jnp.float32,
                  precision=lax.Precision.HIGHEST)
    o_ref[...] = num / (den + 1e-16) + b_ref[...]


def _tc_post(p, q, b):
    return pl.pallas_call(
        _post_body,
        grid=(_NBLK,),
        in_specs=[
            pl.BlockSpec((2, _BLK, ROWW), lambda i: (0, i, 0)),
            pl.BlockSpec((ROWW, CH), lambda i: (0, 0)),
            pl.BlockSpec((1, CH), lambda i: (0, 0)),
        ],
        out_specs=pl.BlockSpec((_BLK, CH), lambda i: (i, 0)),
        out_shape=jax.ShapeDtypeStruct((N, CH), jnp.float32),
    )(p, q, b)


def _make_edge_pass(heads):
    mesh = plsc.VectorSubcoreMesh(core_axis_name="c", subcore_axis_name="s")

    @functools.partial(
        pl.kernel,
        out_type=jax.ShapeDtypeStruct((2, NP_, ROWW), jnp.float32),
        mesh=mesh,
        compiler_params=pltpu.CompilerParams(
            use_tc_tiling_on_sc=False, needs_layout_passes=False),
        scratch_types=[
            pltpu.VMEM_SHARED((NP_, ROWW), jnp.float32),
            [pltpu.VMEM((K,), jnp.int32) for _ in range(NBUF)],
            [pltpu.VMEM((K,), jnp.int32) for _ in range(NBUF)],
            [pltpu.VMEM((K,), jnp.int32) for _ in range(NBUF)],
            [pltpu.VMEM((K, CH), jnp.float32) for _ in range(NBUF)],
            [pltpu.VMEM((K, 16), jnp.float32) for _ in range(NBUF)],
            [pltpu.VMEM((K, 16), jnp.float32) for _ in range(NBUF)],
            [pltpu.VMEM((K, ROWW), jnp.float32) for _ in range(NBUF)],
            [pltpu.SemaphoreType.DMA for _ in range(NBUF)],
            [pltpu.SemaphoreType.DMA for _ in range(NBUF)],
            [pltpu.SemaphoreType.DMA for _ in range(NBUF)],
            [pltpu.SemaphoreType.DMA for _ in range(NBUF)],
            [pltpu.SemaphoreType.DMA for _ in range(NBUF)],
        ],
    )
    def edge_pass(h_hbm, es_hbm, ed_hbm, src_hbm, dst_hbm, z_hbm, out_hbm,
                  acc, src_v, dst_v, sdst, h_rows, es_rows, ed_rows, msg,
                  sem_i, sem_h, sem_e, sem_d, sem_s):
        cid = lax.axis_index("c")
        sid = lax.axis_index("s")
        wid = sid * 2 + cid

        pltpu.sync_copy(z_hbm, acc.at[pl.ds(sid * RPT, RPT)])
        plsc.subcore_barrier()

        e_base = wid * PER_TILE

        def scat_start(b):
            pltpu.async_copy(msg[b], acc.at[sdst[b]], sem_s[b], add=True)

        def scat_drain(b):
            pltpu.make_async_copy(msg[b], acc.at[sdst[b]], sem_s[b]).wait()

        def idx_descs(b, i):
            e0 = e_base + i * K
            return (pltpu.make_async_copy(src_hbm.at[pl.ds(e0, K)],
                                          src_v[b], sem_i[b]),
                    pltpu.make_async_copy(dst_hbm.at[pl.ds(e0, K)],
                                          dst_v[b], sem_i[b]))

        def issue_idx(b, i):
            for d in idx_descs(b, i):
                d.start()

        def issue_gathers(b, i):
            for d in idx_descs(b, i):
                d.wait()
            pltpu.async_copy(h_hbm.at[src_v[b]], h_rows[b], sem_h[b])
            pltpu.async_copy(es_hbm.at[src_v[b]], es_rows[b], sem_e[b])
            pltpu.async_copy(ed_hbm.at[dst_v[b]], ed_rows[b], sem_d[b])

        def consume_pre(b, i):
            @pl.when(i >= 2)
            def _():
                scat_drain(b)

            pltpu.make_async_copy(
                es_hbm.at[src_v[b]], es_rows[b], sem_e[b]).wait()
            pltpu.make_async_copy(
                ed_hbm.at[dst_v[b]], ed_rows[b], sem_d[b]).wait()
            pltpu.make_async_copy(
                h_hbm.at[src_v[b]], h_rows[b], sem_h[b]).wait()

            for j in range(K // 16):
                sdst[b][pl.ds(j * 16, 16)] = dst_v[b][pl.ds(j * 16, 16)]

        def consume_main(b, i):
            mb = msg[b]
            eb = es_rows[b]
            db = ed_rows[b]
            hb = h_rows[b]
            iv = lax.iota(jnp.int32, 16)
            for g in range(K // 16):
                kidx = iv + g * 16
                for hd in range(heads):
                    col = jnp.full((16,), hd, jnp.int32)
                    s = (plsc.load_gather(eb, [kidx, col])
                         + plsc.load_gather(db, [kidx, col]))
                    s = jnp.where(s > 0, s, 0.2 * s)
                    w = jnp.exp(s)
                    plsc.store_scatter(
                        mb, [kidx, jnp.full((16,), DEN0 + hd, jnp.int32)], w)

            @plsc.parallel_loop(0, K, step=1, unroll=8)
            def edge_body(k):
                wv = mb[k, pl.ds(DEN0 - 8, 16)]
                for hd in range(8):
                    ws = wv[8 + (hd % heads)]
                    mb[k, pl.ds(hd * 16, 16)] = ws * hb[k, pl.ds(hd * 16, 16)]

            scat_start(b)

        issue_idx(0, 0)
        issue_gathers(0, 0)
        issue_idx(1, 1)

        def chunk_body(i2, carry):
            for b in range(NBUF):
                i = i2 * NBUF + b
                nb = (b + 1) % NBUF

                consume_pre(b, i)

                @pl.when(i + 1 < CHUNKS)
                def _():
                    issue_gathers(nb, i + 1)

                @pl.when(i + 2 < CHUNKS)
                def _():
                    issue_idx(b, i + 2)

                consume_main(b, i)
            return carry

        lax.fori_loop(0, CHUNKS // NBUF, chunk_body, 0)

        scat_drain((CHUNKS - 2) % NBUF)
        scat_drain((CHUNKS - 1) % NBUF)

        plsc.subcore_barrier()
        pltpu.sync_copy(acc.at[pl.ds(sid * RPT, RPT)],
                        out_hbm.at[cid, pl.ds(sid * RPT, RPT)])

    return edge_pass


_edge_pass8 = _make_edge_pass(NHEAD1)
_edge_pass1 = _make_edge_pass(1)


def kernel(x, edge_index, W1, a_src1, a_dst1, b1, W2, a_src2, a_dst2, b2):
    f32 = jnp.float32
    cols = jnp.arange(CH)
    grp = cols // 16

    asg1 = jnp.zeros((CH, 16), f32).at[cols, grp].set(a_src1.reshape(CH))
    adg1 = jnp.zeros((CH, 16), f32).at[cols, grp].set(a_dst1.reshape(CH))
    asg2 = jnp.zeros((CH, 16), f32).at[:, 0].set(a_src2[0])
    adg2 = jnp.zeros((CH, 16), f32).at[:, 0].set(a_dst2[0])

    q1 = jnp.zeros((ROWW, CH), f32).at[DEN0 + grp, cols].set(1.0)
    q2 = jnp.zeros((ROWW, CH), f32).at[DEN0, cols].set(1.0)

    loop = jnp.arange(N, dtype=jnp.int32)
    npad = EPAD - ET
    src = jnp.concatenate([edge_index[0], loop,
                           jnp.zeros((npad,), jnp.int32)])
    dst = jnp.concatenate([edge_index[1], loop,
                           jnp.full((npad,), N, jnp.int32)])
    zrows = jnp.zeros((RPT, ROWW), f32)
    padrows = jnp.zeros((NP_ - N, 16), f32)

    h1, es1, ed1 = _tc_pre(x, W1, asg1, adg1)
    ed1p = jnp.concatenate([ed1, padrows])
    p1 = _edge_pass8(h1, es1, ed1p, src, dst, zrows)

    h2, es2, ed2 = _tc_mid(p1, q1, b1.reshape(1, CH), W2, asg2, adg2)
    ed2p = jnp.concatenate([ed2, padrows])
    p2 = _edge_pass1(h2, es2, ed2p, src, dst, zrows)

    return _tc_post(p2, q2, b2.reshape(1, CH))
